# Initial kernel scaffold; baseline (speedup 1.0000x reference)
import functools

import jax
import jax.numpy as jnp
from jax import lax
from jax.experimental import pallas as pl
from jax.experimental.pallas import tpu as pltpu
from jax.experimental.pallas import tpu_sc as plsc

NC = 2
NS = 16
NW = NC * NS
L = 16
K = 128

_mesh = plsc.VectorSubcoreMesh(core_axis_name="c", subcore_axis_name="s")


def _deg_call(n, n_pad, chunks, row_r, zdeg, ones_src):
    rows_per_sub_pad = n_pad // NS
    rows_per_sub = n // NS

    @functools.partial(
        pl.kernel,
        out_type=jax.ShapeDtypeStruct((NC, n, L), jnp.float32),
        mesh=_mesh,
        scratch_types=[
            pltpu.VMEM((chunks, K), jnp.int32),
            pltpu.VMEM((K, L), jnp.float32),
            pltpu.VMEM_SHARED((n_pad, L), jnp.float32),
        ],
    )
    def deg_kernel(row_hbm, z_hbm, ones_hbm, out_hbm, idx_v, ones_v, acc):
        cid = lax.axis_index("c")
        sid = lax.axis_index("s")
        wid = sid * NC + cid
        pltpu.sync_copy(ones_hbm, ones_v)
        pltpu.sync_copy(z_hbm.at[pl.ds(sid * rows_per_sub_pad, rows_per_sub_pad)],
                        acc.at[pl.ds(sid * rows_per_sub_pad, rows_per_sub_pad)])
        pltpu.sync_copy(row_hbm.at[wid], idx_v)
        plsc.subcore_barrier()

        @pl.loop(0, chunks)
        def _(j):
            pltpu.sync_copy(ones_v, acc.at[idx_v.at[j]], add=True)

        plsc.subcore_barrier()
        pltpu.sync_copy(acc.at[pl.ds(sid * rows_per_sub, rows_per_sub)],
                        out_hbm.at[cid, pl.ds(sid * rows_per_sub, rows_per_sub)])

    return deg_kernel(row_r, zdeg, ones_src)


def _prop_call(n, n_pad, chunks, d, h2, row_r, col_r, zmain):
    rows_per_sub_pad = n_pad // NS
    rows_per_sub = n // NS

    @functools.partial(
        pl.kernel,
        out_type=jax.ShapeDtypeStruct((NC, n, d), jnp.float32),
        mesh=_mesh,
        scratch_types=[
            pltpu.VMEM((chunks, K), jnp.int32),
            pltpu.VMEM((chunks, K), jnp.int32),
            pltpu.VMEM((K, d), jnp.float32),
            pltpu.VMEM_SHARED((n_pad, d), jnp.float32),
            pltpu.SemaphoreType.DMA,
        ],
    )
    def prop_kernel(h2_hbm, row_hbm, col_hbm, z_hbm, out_hbm,
                    ri_v, ci_v, buf, acc, sem):
        cid = lax.axis_index("c")
        sid = lax.axis_index("s")
        wid = sid * NC + cid
        pltpu.sync_copy(z_hbm.at[pl.ds(sid * rows_per_sub_pad, rows_per_sub_pad)],
                        acc.at[pl.ds(sid * rows_per_sub_pad, rows_per_sub_pad)])
        pltpu.sync_copy(row_hbm.at[wid], ri_v)
        pltpu.sync_copy(col_hbm.at[wid], ci_v)
        plsc.subcore_barrier()

        @pl.loop(0, chunks)
        def _(j):
            pltpu.async_copy(h2_hbm.at[ri_v.at[j]], buf, sem).wait()
            pltpu.sync_copy(buf, acc.at[ci_v.at[j]], add=True)

        plsc.subcore_barrier()
        pltpu.sync_copy(acc.at[pl.ds(sid * rows_per_sub, rows_per_sub)],
                        out_hbm.at[cid, pl.ds(sid * rows_per_sub, rows_per_sub)])

    return prop_kernel(h2, row_r, col_r, zmain)


def _matmul_call(x, w, dega, degb, block_rows):
    n, d = x.shape
    u = w.shape[1]

    def body(x_ref, w_ref, da_ref, db_ref, h2_ref):
        deg = da_ref[:, 0:1] + db_ref[:, 0:1] + 1.0
        h = jnp.dot(x_ref[...], w_ref[...], preferred_element_type=jnp.float32)
        h2_ref[...] = h * lax.rsqrt(deg)

    return pl.pallas_call(
        body,
        grid=(n // block_rows,),
        in_specs=[
            pl.BlockSpec((block_rows, d), lambda i: (i, 0)),
            pl.BlockSpec((d, u), lambda i: (0, 0)),
            pl.BlockSpec((block_rows, L), lambda i: (i, 0)),
            pl.BlockSpec((block_rows, L), lambda i: (i, 0)),
        ],
        out_specs=pl.BlockSpec((block_rows, u), lambda i: (i, 0)),
        out_shape=jax.ShapeDtypeStruct((n, u), jnp.float32),
    )(x, w, dega, degb)


def _combine_call(p0, p1, h2, dega, degb, bias2d, block_rows):
    n, u = h2.shape

    def body(p0_ref, p1_ref, h2_ref, da_ref, db_ref, b_ref, o_ref):
        dinv = lax.rsqrt(da_ref[:, 0:1] + db_ref[:, 0:1] + 1.0)
        o_ref[...] = (p0_ref[...] + p1_ref[...] + h2_ref[...]) * dinv + b_ref[...]

    return pl.pallas_call(
        body,
        grid=(n // block_rows,),
        in_specs=[
            pl.BlockSpec((block_rows, u), lambda i: (i, 0)),
            pl.BlockSpec((block_rows, u), lambda i: (i, 0)),
            pl.BlockSpec((block_rows, u), lambda i: (i, 0)),
            pl.BlockSpec((block_rows, L), lambda i: (i, 0)),
            pl.BlockSpec((block_rows, L), lambda i: (i, 0)),
            pl.BlockSpec((1, u), lambda i: (0, 0)),
        ],
        out_specs=pl.BlockSpec((block_rows, u), lambda i: (i, 0)),
        out_shape=jax.ShapeDtypeStruct((n, u), jnp.float32),
    )(p0, p1, h2, dega, degb, bias2d)


def kernel(x, edge_index, kernel, bias):
    n, d = x.shape
    u = kernel.shape[1]
    e = edge_index.shape[1]

    chunks = -(-e // (NW * K))
    e_pad = NW * chunks * K
    pad = e_pad - e
    n_pad = -(-(n + 1) // NS) * NS

    row = edge_index[0]
    col = edge_index[1]
    row_deg = jnp.concatenate(
        [row, jnp.full((pad,), n, jnp.int32)]).reshape(NW, chunks, K)
    row_main = jnp.concatenate(
        [row, jnp.zeros((pad,), jnp.int32)]).reshape(NW, chunks, K)
    col_main = jnp.concatenate(
        [col, jnp.full((pad,), n, jnp.int32)]).reshape(NW, chunks, K)

    zdeg = jnp.zeros((n_pad, L), jnp.float32)
    zmain = jnp.zeros((n_pad, d), jnp.float32)
    ones_src = jnp.ones((K, L), jnp.float32)

    degp = _deg_call(n, n_pad, chunks, row_deg, zdeg, ones_src)
    dega = degp[0]
    degb = degp[1]

    h2 = _matmul_call(x, kernel, dega, degb, block_rows=1000)

    p = _prop_call(n, n_pad, chunks, d, h2, row_main, col_main, zmain)

    out = _combine_call(p[0], p[1], h2, dega, degb,
                        bias.reshape(1, u), block_rows=1000)
    return out


# speedup vs baseline: 19.4857x; 19.4857x over previous
import dataclasses
import functools

import jax
import jax.numpy as jnp
from jax import lax
from jax.experimental import pallas as pl
from jax.experimental.pallas import tpu as pltpu
from jax.experimental.pallas import tpu_sc as plsc

NC = 2
NS = 16
NW = NC * NS
L = 16
K = 128

_mesh = plsc.VectorSubcoreMesh(core_axis_name="c", subcore_axis_name="s")
_sc_params = dataclasses.replace(pltpu.CompilerParams(),
                                 needs_layout_passes=False)


def _deg_call(n_pad, epw, row_flat, zdeg):
    @functools.partial(
        pl.kernel,
        out_type=jax.ShapeDtypeStruct((NW, n_pad), jnp.float32),
        mesh=_mesh,
        scratch_types=[
            pltpu.VMEM((epw,), jnp.int32),
            pltpu.VMEM((n_pad,), jnp.float32),
        ],
        compiler_params=_sc_params,
    )
    def deg_kernel(row_hbm, z_hbm, out_hbm, idx_v, deg_v):
        cid = lax.axis_index("c")
        sid = lax.axis_index("s")
        wid = sid * NC + cid
        pltpu.sync_copy(z_hbm, deg_v)
        pltpu.sync_copy(row_hbm.at[wid], idx_v)
        ones = jnp.ones((L,), jnp.float32)

        @pl.loop(0, epw // L)
        def _(v):
            iv = idx_v[pl.ds(v * L, L)]
            plsc.addupdate_scatter(deg_v, [iv], ones)

        pltpu.sync_copy(deg_v, out_hbm.at[wid])

    return deg_kernel(row_flat, zdeg)


def _prop_call(n_pad, chunks, d, h2, row_r, col_r, zmain):
    rows_per_sub = n_pad // NS

    @functools.partial(
        pl.kernel,
        out_type=jax.ShapeDtypeStruct((NC, n_pad, d), jnp.float32),
        mesh=_mesh,
        scratch_types=[
            pltpu.VMEM((chunks, K), jnp.int32),
            pltpu.VMEM((chunks, K), jnp.int32),
            pltpu.VMEM((K, d), jnp.float32),
            pltpu.VMEM_SHARED((n_pad, d), jnp.float32),
            pltpu.SemaphoreType.DMA,
        ],
    )
    def prop_kernel(h2_hbm, row_hbm, col_hbm, z_hbm, out_hbm,
                    ri_v, ci_v, buf, acc, sem):
        cid = lax.axis_index("c")
        sid = lax.axis_index("s")
        wid = sid * NC + cid
        pltpu.sync_copy(z_hbm.at[pl.ds(sid * rows_per_sub, rows_per_sub)],
                        acc.at[pl.ds(sid * rows_per_sub, rows_per_sub)])
        pltpu.sync_copy(row_hbm.at[wid], ri_v)
        pltpu.sync_copy(col_hbm.at[wid], ci_v)
        plsc.subcore_barrier()

        @pl.loop(0, chunks)
        def _(j):
            pltpu.async_copy(h2_hbm.at[ri_v.at[j]], buf, sem).wait()
            pltpu.sync_copy(buf, acc.at[ci_v.at[j]], add=True)

        plsc.subcore_barrier()
        pltpu.sync_copy(acc.at[pl.ds(sid * rows_per_sub, rows_per_sub)],
                        out_hbm.at[cid, pl.ds(sid * rows_per_sub, rows_per_sub)])

    return prop_kernel(h2, row_r, col_r, zmain)


def _matmul_call(x, w, degt, block_rows):
    n, d = x.shape
    u = w.shape[1]

    def body(x_ref, w_ref, dt_ref, h2_ref):
        deg = jnp.sum(dt_ref[...], axis=1, keepdims=True) + 1.0
        h = jnp.dot(x_ref[...], w_ref[...], preferred_element_type=jnp.float32)
        h2_ref[...] = h * lax.rsqrt(deg)

    return pl.pallas_call(
        body,
        grid=(n // block_rows,),
        in_specs=[
            pl.BlockSpec((block_rows, d), lambda i: (i, 0)),
            pl.BlockSpec((d, u), lambda i: (0, 0)),
            pl.BlockSpec((block_rows, NW), lambda i: (i, 0)),
        ],
        out_specs=pl.BlockSpec((block_rows, u), lambda i: (i, 0)),
        out_shape=jax.ShapeDtypeStruct((n, u), jnp.float32),
    )(x, w, degt)


def _combine_call(p0, p1, h2, degt, bias2d, block_rows):
    n, u = h2.shape

    def body(p0_ref, p1_ref, h2_ref, dt_ref, b_ref, o_ref):
        deg = jnp.sum(dt_ref[...], axis=1, keepdims=True) + 1.0
        dinv = lax.rsqrt(deg)
        o_ref[...] = (p0_ref[...] + p1_ref[...] + h2_ref[...]) * dinv + b_ref[...]

    return pl.pallas_call(
        body,
        grid=(n // block_rows,),
        in_specs=[
            pl.BlockSpec((block_rows, u), lambda i: (i, 0)),
            pl.BlockSpec((block_rows, u), lambda i: (i, 0)),
            pl.BlockSpec((block_rows, u), lambda i: (i, 0)),
            pl.BlockSpec((block_rows, NW), lambda i: (i, 0)),
            pl.BlockSpec((1, u), lambda i: (0, 0)),
        ],
        out_specs=pl.BlockSpec((block_rows, u), lambda i: (i, 0)),
        out_shape=jax.ShapeDtypeStruct((n, u), jnp.float32),
    )(p0, p1, h2, degt, bias2d)


def kernel(x, edge_index, kernel, bias):
    n, d = x.shape
    u = kernel.shape[1]
    e = edge_index.shape[1]

    chunks = -(-e // (NW * K))
    epw = chunks * K
    pad = NW * epw - e
    n_pad = -(-(n + 1) // (NS * 8)) * (NS * 8)

    row = edge_index[0]
    col = edge_index[1]
    row_deg = jnp.concatenate(
        [row, jnp.full((pad,), n, jnp.int32)]).reshape(NW, epw)
    row_main = jnp.concatenate(
        [row, jnp.zeros((pad,), jnp.int32)]).reshape(NW, chunks, K)
    col_main = jnp.concatenate(
        [col, jnp.full((pad,), n, jnp.int32)]).reshape(NW, chunks, K)

    zdeg = jnp.zeros((n_pad,), jnp.float32)
    zmain = jnp.zeros((n_pad, d), jnp.float32)

    degp = _deg_call(n_pad, epw, row_deg, zdeg)
    degt = degp.T[:n]

    h2 = _matmul_call(x, kernel, degt, block_rows=1000)

    p = _prop_call(n_pad, chunks, d, h2, row_main, col_main, zmain)

    out = _combine_call(p[0, :n], p[1, :n], h2, degt,
                        bias.reshape(1, u), block_rows=1000)
    return out
